# Initial kernel scaffold; baseline (speedup 1.0000x reference)
import jax
import jax.numpy as jnp
from jax.experimental import pallas as pl


def kernel(x, edge_index, W1, b1, W2, b2):
    raise NotImplementedError("write your pallas kernel here")


# speedup vs baseline: 6.0082x; 6.0082x over previous
import functools

import jax
import jax.numpy as jnp
from jax import lax
from jax.experimental import pallas as pl
from jax.experimental.pallas import tpu as pltpu
from jax.experimental.pallas import tpu_sc as plsc

N = 10000
E = 320000
D = 128

NC, NS, L = 2, 16, 16
TILES = NC * NS
NP = 10240
ROWS_PT = NP // NS
NR = NP // 128
CHUNK = 128
CH = 80
EP = TILES * CH * CHUNK
PAD_ROW = N

_mesh = plsc.VectorSubcoreMesh(
    core_axis_name="c", subcore_axis_name="s", num_cores=NC, num_subcores=NS)
_sc_params = pltpu.CompilerParams(needs_layout_passes=False)


@functools.partial(
    pl.kernel,
    out_type=jax.ShapeDtypeStruct((TILES * NP,), jnp.float32),
    mesh=_mesh,
    compiler_params=_sc_params,
    scratch_types=[
        pltpu.VMEM((CH, CHUNK), jnp.int32),
        pltpu.VMEM((NP,), jnp.float32),
    ],
)
def _sc_degree(dst_hbm, deg_out, idx_v, hist_v):
    cid = lax.axis_index("c")
    sid = lax.axis_index("s")
    wid = cid * NS + sid

    pltpu.sync_copy(dst_hbm.at[wid], idx_v)

    zeros16 = jnp.zeros((L,), jnp.float32)
    ones16 = jnp.ones((L,), jnp.float32)

    def _zero(i, _):
        hist_v[pl.ds(i * L, L)] = zeros16
        return _
    lax.fori_loop(0, NP // L, _zero, None)

    def _accum(j, _):
        for k in range(CHUNK // L):
            d = idx_v[j, pl.ds(k * L, L)]
            plsc.addupdate_scatter(hist_v, [d], ones16)
        return _
    lax.fori_loop(0, CH, _accum, None)

    pltpu.sync_copy(hist_v, deg_out.at[pl.ds(wid * NP, NP)])


GD = 8
NG = CH // GD


@functools.partial(
    pl.kernel,
    out_type=jax.ShapeDtypeStruct((NC * NP, D), jnp.float32),
    mesh=_mesh,
    compiler_params=_sc_params,
    scratch_types=[
        pltpu.VMEM((CH + 2, CHUNK), jnp.int32),
        pltpu.VMEM((GD, CHUNK), jnp.int32),
        pltpu.VMEM((CHUNK, D), jnp.float32),
        pltpu.VMEM((CHUNK, D), jnp.float32),
        pltpu.VMEM_SHARED((NP, D), jnp.float32),
        pltpu.SemaphoreType.DMA,
        pltpu.SemaphoreType.DMA,
    ],
)
def _sc_scatter(src_hbm, dst_hbm, y_hbm, z_out,
                isrc_v, idst_v, buf0, buf1, z_sh, sem0, sem1):
    cid = lax.axis_index("c")
    sid = lax.axis_index("s")
    wid = cid * NS + sid

    pltpu.sync_copy(src_hbm.at[wid], isrc_v.at[pl.ds(0, CH)])
    zeros16i = jnp.zeros((L,), jnp.int32)
    for k in range(CHUNK // L):
        isrc_v[CH, pl.ds(k * L, L)] = zeros16i
        isrc_v[CH + 1, pl.ds(k * L, L)] = zeros16i

    zeros16 = jnp.zeros((L,), jnp.float32)

    def _zero_row(r, _):
        for k in range(D // L):
            buf0[r, pl.ds(k * L, L)] = zeros16
        return _
    lax.fori_loop(0, CHUNK, _zero_row, None)
    for c in range(ROWS_PT // CHUNK):
        pltpu.sync_copy(buf0, z_sh.at[pl.ds(sid * ROWS_PT + c * CHUNK, CHUNK)])
    plsc.subcore_barrier()

    def _start(j, buf, sem):
        pltpu.async_copy(y_hbm.at[isrc_v.at[j]], buf, sem)

    def _wait(buf, sem):
        pltpu.make_async_copy(y_hbm.at[isrc_v.at[0]], buf, sem).wait()

    _start(0, buf0, sem0)
    _start(1, buf1, sem1)

    def _group(g, _):
        pltpu.sync_copy(dst_hbm.at[wid, pl.ds(g * GD, GD)], idst_v)
        for jj in range(GD):
            j = g * GD + jj
            buf, sem = (buf0, sem0) if jj % 2 == 0 else (buf1, sem1)
            _wait(buf, sem)
            pltpu.sync_copy(buf, z_sh.at[idst_v.at[jj]], add=True)
            _start(j + 2, buf, sem)
        return _
    lax.fori_loop(0, NG, _group, None)
    _wait(buf0, sem0)
    _wait(buf1, sem1)
    plsc.subcore_barrier()

    pltpu.sync_copy(z_sh.at[pl.ds(sid * ROWS_PT, ROWS_PT)],
                    z_out.at[pl.ds(cid * NP + sid * ROWS_PT, ROWS_PT)])


_RB = 512
_GRID = NP // _RB


def _tc_in_body(deg_ref, x_ref, w_ref, y_ref, dinv_ref):
    deg = jnp.sum(deg_ref[...], axis=1, keepdims=True) + 1.0
    dinv = lax.rsqrt(deg)
    y_ref[...] = dinv * jnp.dot(x_ref[...], w_ref[...],
                                preferred_element_type=jnp.float32)
    dinv_ref[...] = dinv


_tc_in = pl.pallas_call(
    _tc_in_body,
    grid=(_GRID,),
    in_specs=[
        pl.BlockSpec((_RB, TILES), lambda i: (i, 0)),
        pl.BlockSpec((_RB, D), lambda i: (i, 0)),
        pl.BlockSpec((D, D), lambda i: (0, 0)),
    ],
    out_specs=[
        pl.BlockSpec((_RB, D), lambda i: (i, 0)),
        pl.BlockSpec((_RB, 1), lambda i: (i, 0)),
    ],
    out_shape=[
        jax.ShapeDtypeStruct((NP, D), jnp.float32),
        jax.ShapeDtypeStruct((NP, 1), jnp.float32),
    ],
)


def _tc_mid_body(z_ref, y1_ref, dinv_ref, w_ref, b_ref, y2_ref):
    dinv = dinv_ref[...]
    s = z_ref[0] + z_ref[1] + y1_ref[...]
    h = jnp.maximum(dinv * s + b_ref[...], 0.0)
    y2_ref[...] = dinv * jnp.dot(h, w_ref[...],
                                 preferred_element_type=jnp.float32)


_tc_mid = pl.pallas_call(
    _tc_mid_body,
    grid=(_GRID,),
    in_specs=[
        pl.BlockSpec((2, _RB, D), lambda i: (0, i, 0)),
        pl.BlockSpec((_RB, D), lambda i: (i, 0)),
        pl.BlockSpec((_RB, 1), lambda i: (i, 0)),
        pl.BlockSpec((D, D), lambda i: (0, 0)),
        pl.BlockSpec((1, D), lambda i: (0, 0)),
    ],
    out_specs=pl.BlockSpec((_RB, D), lambda i: (i, 0)),
    out_shape=jax.ShapeDtypeStruct((NP, D), jnp.float32),
)


def _tc_out_body(z_ref, y2_ref, dinv_ref, b_ref, o_ref):
    s = z_ref[0] + z_ref[1] + y2_ref[...]
    o_ref[...] = dinv_ref[...] * s + b_ref[...]


_tc_out = pl.pallas_call(
    _tc_out_body,
    grid=(_GRID,),
    in_specs=[
        pl.BlockSpec((2, _RB, D), lambda i: (0, i, 0)),
        pl.BlockSpec((_RB, D), lambda i: (i, 0)),
        pl.BlockSpec((_RB, 1), lambda i: (i, 0)),
        pl.BlockSpec((1, D), lambda i: (0, 0)),
    ],
    out_specs=pl.BlockSpec((_RB, D), lambda i: (i, 0)),
    out_shape=jax.ShapeDtypeStruct((NP, D), jnp.float32),
)


def kernel(x, edge_index, W1, b1, W2, b2):
    ei = edge_index.astype(jnp.int32)
    pad = EP - E
    src = jnp.concatenate([ei[0], jnp.zeros((pad,), jnp.int32)])
    dst = jnp.concatenate([ei[1], jnp.full((pad,), PAD_ROW, jnp.int32)])
    src3 = src.reshape(TILES, CH, CHUNK)
    dst3 = dst.reshape(TILES, CH, CHUNK)
    x_pad = jnp.pad(x, ((0, NP - N), (0, 0)))

    deg2 = _sc_degree(dst3)
    deg_t = deg2.reshape(TILES, NP).T

    y1, dinv = _tc_in(deg_t, x_pad, W1)
    z1 = _sc_scatter(src3, dst3, y1).reshape(2, NP, D)
    y2 = _tc_mid(z1, y1, dinv, W2, b1.reshape(1, D))
    z2 = _sc_scatter(src3, dst3, y2).reshape(2, NP, D)
    out = _tc_out(z2, y2, dinv, b2.reshape(1, D))
    return out[:N]
